# Initial kernel scaffold; baseline (speedup 1.0000x reference)
import jax
import jax.numpy as jnp
from jax.experimental import pallas as pl


def kernel(x, edge_index0, edge_index1, W_l0, W_r0, b0, W_l1, W_r1, b1):
    raise NotImplementedError("write your pallas kernel here")


# speedup vs baseline: 6.6389x; 6.6389x over previous
import dataclasses
import functools

import jax
import jax.numpy as jnp
from jax import lax
from jax.experimental import pallas as pl
from jax.experimental.pallas import tpu as pltpu
from jax.experimental.pallas import tpu_sc as plsc

N0, N1, N2 = 10000, 4096, 1024
E0, E1 = 320000, 131072
D = 128
NC, NS = 2, 16
NW = NC * NS


def _make_sc_agg(n_tgt, n_edges, chunk):
    n_chunks = n_edges // (NW * chunk)
    assert n_chunks * chunk * NW == n_edges
    assert chunk % 8 == 0 and chunk <= 128
    per_w = chunk * n_chunks
    rows_per = n_tgt // NS
    mesh = plsc.VectorSubcoreMesh(core_axis_name="c", subcore_axis_name="s")
    cp = pltpu.CompilerParams()
    if "needs_layout_passes" in pltpu.CompilerParams.__dataclass_fields__:
        cp = dataclasses.replace(cp, needs_layout_passes=False)

    @functools.partial(
        pl.kernel,
        compiler_params=cp,
        out_type=(jax.ShapeDtypeStruct((NC, n_tgt, D), jnp.float32),
                  jax.ShapeDtypeStruct((NW * n_tgt,), jnp.float32)),
        mesh=mesh,
        scratch_types=[
            pltpu.VMEM((chunk,), jnp.int32),
            pltpu.VMEM((chunk,), jnp.int32),
            pltpu.VMEM((chunk, D), jnp.float32),
            pltpu.VMEM((n_tgt,), jnp.float32),
            pltpu.VMEM_SHARED((n_tgt, D), jnp.float32),
            pltpu.SemaphoreType.DMA,
        ],
    )
    def agg(x_hbm, src_hbm, dst_hbm, zs_hbm, zc_hbm, sum_hbm, cnt_hbm,
            src_v, dst_v, rows_v, cnt_v, acc_sh, sem):
        c = lax.axis_index("c")
        s = lax.axis_index("s")
        wid = c * NS + s

        pltpu.sync_copy(zc_hbm, cnt_v)
        r0 = s * rows_per
        pltpu.sync_copy(zs_hbm.at[pl.ds(r0, rows_per)],
                        acc_sh.at[pl.ds(r0, rows_per)])
        plsc.subcore_barrier()

        base_w = wid * per_w
        ones16 = jnp.ones((16,), jnp.float32)

        @pl.loop(0, n_chunks)
        def _(j):
            base = base_w + j * chunk
            pltpu.sync_copy(src_hbm.at[pl.ds(base, chunk)], src_v)
            pltpu.sync_copy(dst_hbm.at[pl.ds(base, chunk)], dst_v)
            pltpu.async_copy(x_hbm.at[src_v], rows_v, sem).wait()
            pltpu.sync_copy(rows_v, acc_sh.at[dst_v], add=True)
            for k in range(chunk // 16):
                didx = dst_v[pl.ds(k * 16, 16)]
                plsc.addupdate_scatter(cnt_v, [didx], ones16)

        plsc.subcore_barrier()
        pltpu.sync_copy(acc_sh.at[pl.ds(r0, rows_per)],
                        sum_hbm.at[c, pl.ds(r0, rows_per)])
        pltpu.sync_copy(cnt_v, cnt_hbm.at[pl.ds(wid * n_tgt, n_tgt)])

    return agg


def _dense_body(relu, logsm):
    def body(s_ref, c_ref, x_ref, wl_ref, wr_ref, b_ref, o_ref):
        ssum = s_ref[0] + s_ref[1]
        cnt_bcast = lax.dot_general(
            c_ref[...], jnp.ones((NW, D), jnp.float32),
            (((0,), (0,)), ((), ())),
            preferred_element_type=jnp.float32,
            precision=lax.Precision.HIGHEST)
        mean = ssum / jnp.maximum(cnt_bcast, 1.0)
        z = (lax.dot_general(mean, wl_ref[...], (((1,), (1,)), ((), ())),
                             preferred_element_type=jnp.float32,
                             precision=lax.Precision.HIGHEST)
             + lax.dot_general(x_ref[...], wr_ref[...], (((1,), (1,)), ((), ())),
                               preferred_element_type=jnp.float32,
                               precision=lax.Precision.HIGHEST)
             + b_ref[...])
        if relu:
            z = jnp.maximum(z, 0.0)
        if logsm:
            m = jnp.max(z, axis=-1, keepdims=True)
            z = z - m - jnp.log(jnp.sum(jnp.exp(z - m), axis=-1, keepdims=True))
        o_ref[...] = z
    return body


def _dense(relu, logsm, n_tgt, sums, cnts, xt, wl, wr, b):
    return pl.pallas_call(
        _dense_body(relu, logsm),
        out_shape=jax.ShapeDtypeStruct((n_tgt, D), jnp.float32),
    )(sums, cnts, xt, wl, wr, b.reshape(1, D))


_agg0 = _make_sc_agg(N1, E0, 80)
_agg1 = _make_sc_agg(N2, E1, 128)


def kernel(x, edge_index0, edge_index1, W_l0, W_r0, b0, W_l1, W_r1, b1):
    src0 = edge_index0[0].astype(jnp.int32)
    dst0 = edge_index0[1].astype(jnp.int32)
    src1 = edge_index1[0].astype(jnp.int32)
    dst1 = edge_index1[1].astype(jnp.int32)

    zs0 = jnp.zeros((N1, D), jnp.float32)
    zc0 = jnp.zeros((N1,), jnp.float32)
    sums0, cnts0 = _agg0(x, src0, dst0, zs0, zc0)
    h = _dense(True, False, N1, sums0, cnts0.reshape(NW, N1), x[:N1],
               W_l0, W_r0, b0)

    zs1 = jnp.zeros((N2, D), jnp.float32)
    zc1 = jnp.zeros((N2,), jnp.float32)
    sums1, cnts1 = _agg1(h, src1, dst1, zs1, zc1)
    out = _dense(False, True, N2, sums1, cnts1.reshape(NW, N2), h[:N2],
                 W_l1, W_r1, b1)
    return out
